# baseline (speedup 1.0000x reference)
import functools

import jax
import jax.numpy as jnp
from jax import lax
from jax.experimental import pallas as pl
from jax.experimental.pallas import tpu as pltpu
from jax.experimental.pallas import tpu_sc as plsc

T = 32768
D_IN = 12
K = 1024
D = 64
DP = 16
W = 128
TB = 1024
NC = 2
NS = 16
NW = NC * NS
BPW = T // NW
G = 128
NG = BPW // G
HALF = NG // 2


def _tc_body(x_ref, wenc_ref, benc_ref, cbt_ref, cb_ref, wd_ref, bd_ref,
             ze_ref, ids_ref, tab_ref):
    i = pl.program_id(0)

    z_e = jnp.dot(x_ref[...], wenc_ref[...],
                  preferred_element_type=jnp.float32) + benc_ref[...]
    ze_ref[...] = z_e

    cbt = cbt_ref[...]
    z2 = jnp.sum(z_e * z_e, axis=1, keepdims=True)
    zc = jnp.dot(z_e, cbt, preferred_element_type=jnp.float32)
    c2 = jnp.sum(cbt * cbt, axis=0, keepdims=True)
    d2 = (z2 - 2.0 * zc) + c2

    m = jnp.min(d2, axis=1, keepdims=True)
    iota = lax.broadcasted_iota(jnp.int32, (TB, K), 1)
    ids = jnp.min(jnp.where(d2 == m, iota, K), axis=1, keepdims=True)
    ids_ref[...] = ids

    @pl.when(i == 0)
    def _():
        cb = cb_ref[...]
        tab_ref[:, 0:D] = cb
        tab_ref[:, D:D + DP] = jnp.dot(cb, wd_ref[...],
                                       preferred_element_type=jnp.float32) + bd_ref[...]
        tab_ref[:, D + DP:W] = jnp.zeros((K, W - D - DP), jnp.float32)


_tc_call = pl.pallas_call(
    _tc_body,
    grid=(T // TB,),
    in_specs=[
        pl.BlockSpec((TB, D_IN), lambda i: (i, 0)),
        pl.BlockSpec((D_IN, D), lambda i: (0, 0)),
        pl.BlockSpec((1, D), lambda i: (0, 0)),
        pl.BlockSpec((D, K), lambda i: (0, 0)),
        pl.BlockSpec((K, D), lambda i: (0, 0)),
        pl.BlockSpec((D, DP), lambda i: (0, 0)),
        pl.BlockSpec((1, DP), lambda i: (0, 0)),
    ],
    out_specs=[
        pl.BlockSpec((TB, D), lambda i: (i, 0)),
        pl.BlockSpec((TB, 1), lambda i: (i, 0)),
        pl.BlockSpec((K, W), lambda i: (0, 0)),
    ],
    out_shape=[
        jax.ShapeDtypeStruct((T, D), jnp.float32),
        jax.ShapeDtypeStruct((T, 1), jnp.int32),
        jax.ShapeDtypeStruct((K, W), jnp.float32),
    ],
)


@functools.partial(
    pl.kernel,
    out_type=jax.ShapeDtypeStruct((T, W), jnp.float32),
    mesh=plsc.VectorSubcoreMesh(core_axis_name="c", subcore_axis_name="s"),
    scratch_types=[
        pltpu.VMEM((NG, G), jnp.int32),
        pltpu.VMEM((HALF * G, W), jnp.float32),
        pltpu.SemaphoreType.DMA,
    ],
)
def _sc_gather(tab_hbm, ids_hbm, out_hbm, idx_v, buf, sem):
    wid = lax.axis_index("s") * NC + lax.axis_index("c")
    base = wid * BPW
    pltpu.sync_copy(ids_hbm.at[pl.ds(wid * NG, NG)], idx_v)
    for r in range(2):
        for j in range(HALF):
            g = r * HALF + j
            pltpu.async_copy(tab_hbm.at[idx_v.at[g]],
                             buf.at[pl.ds(j * G, G)], sem)
        for j in range(HALF):
            pltpu.make_async_copy(tab_hbm.at[idx_v.at[r * HALF + j]],
                                  buf.at[pl.ds(j * G, G)], sem).wait()
        pltpu.sync_copy(buf, out_hbm.at[pl.ds(base + r * HALF * G, HALF * G)])


def kernel(x, W_enc, b_enc, codebook, W_dec, b_dec):
    wd_pad = jnp.zeros((D, DP), jnp.float32).at[:, :D_IN].set(W_dec)
    bd_pad = jnp.zeros((1, DP), jnp.float32).at[0, :D_IN].set(b_dec)
    z_e, ids2d, tab = _tc_call(
        x, W_enc, b_enc.reshape(1, D), codebook.T, codebook, wd_pad, bd_pad)
    ids = ids2d.reshape(T // G, G)
    out = _sc_gather(tab, ids)
    return (out[:, D:D + D_IN], z_e, out[:, :D])


# speedup vs baseline: 2.4217x; 2.4217x over previous
import functools

import jax
import jax.numpy as jnp
from jax import lax
from jax.experimental import pallas as pl
from jax.experimental.pallas import tpu as pltpu
from jax.experimental.pallas import tpu_sc as plsc

T = 32768
D_IN = 12
K = 1024
D = 64
DP = 16
TW = D + DP
TB = 1024
NC = 2
NS = 16
NW = NC * NS
BPW = T // NW
CH = 128
L = 16


def _tc_body(x_ref, wenc_ref, benc_ref, cbt_ref, cb_ref, wd_ref, bd_ref,
             ze_ref, ids_ref, tab_ref, c2_ref):
    i = pl.program_id(0)

    @pl.when(i == 0)
    def _():
        cbt = cbt_ref[...]
        c2_ref[...] = jnp.sum(cbt * cbt, axis=0, keepdims=True)
        cb = cb_ref[...]
        tab_ref[:, 0:D] = cb
        tab_ref[:, D:TW] = jnp.dot(cb, wd_ref[...],
                                   preferred_element_type=jnp.float32) + bd_ref[...]

    x = x_ref[...]
    z_e = jnp.dot(x, wenc_ref[...],
                  preferred_element_type=jnp.float32) + benc_ref[...]
    ze_ref[...] = z_e

    z2 = jnp.sum(z_e * z_e, axis=1, keepdims=True)
    zc = jnp.dot(z_e, cbt_ref[...],
                 preferred_element_type=jnp.float32)
    d2 = (z2 - 2.0 * zc) + c2_ref[...]

    m = jnp.min(d2, axis=1, keepdims=True)
    iota = lax.broadcasted_iota(jnp.int32, (TB, K), 1)
    ids_ref[...] = jnp.min(jnp.where(d2 == m, iota, K), axis=1, keepdims=True)


_tc_call = pl.pallas_call(
    _tc_body,
    grid=(T // TB,),
    in_specs=[
        pl.BlockSpec((TB, D_IN), lambda i: (i, 0)),
        pl.BlockSpec((D_IN, D), lambda i: (0, 0)),
        pl.BlockSpec((1, D), lambda i: (0, 0)),
        pl.BlockSpec((D, K), lambda i: (0, 0)),
        pl.BlockSpec((K, D), lambda i: (0, 0)),
        pl.BlockSpec((D, DP), lambda i: (0, 0)),
        pl.BlockSpec((1, DP), lambda i: (0, 0)),
    ],
    out_specs=[
        pl.BlockSpec((TB, D), lambda i: (i, 0)),
        pl.BlockSpec((TB, 1), lambda i: (i, 0)),
        pl.BlockSpec((K, TW), lambda i: (0, 0)),
    ],
    out_shape=[
        jax.ShapeDtypeStruct((T, D), jnp.float32),
        jax.ShapeDtypeStruct((T, 1), jnp.int32),
        jax.ShapeDtypeStruct((K, TW), jnp.float32),
    ],
    scratch_shapes=[
        pltpu.VMEM((1, K), jnp.float32),
    ],
)


@functools.partial(
    pl.kernel,
    out_type=(jax.ShapeDtypeStruct((T, D), jnp.float32),
              jax.ShapeDtypeStruct((T, DP), jnp.float32)),
    mesh=plsc.VectorSubcoreMesh(core_axis_name="c", subcore_axis_name="s"),
    scratch_types=[
        pltpu.VMEM((K * TW,), jnp.float32),
        pltpu.VMEM((BPW,), jnp.int32),
        pltpu.VMEM((CH, D), jnp.float32),
        pltpu.VMEM((CH, DP), jnp.float32),
    ],
    compiler_params=pltpu.CompilerParams(needs_layout_passes=False),
)
def _sc_gather(tab_hbm, ids_hbm, zq_hbm, xr_hbm, tab_v, ids_v, zqb, xrb):
    wid = lax.axis_index("s") * NC + lax.axis_index("c")
    base = wid * BPW
    pltpu.sync_copy(tab_hbm, tab_v)
    pltpu.sync_copy(ids_hbm.at[pl.ds(base, BPW)], ids_v)
    lanes = lax.iota(jnp.int32, L)
    zero16 = jnp.zeros((L,), jnp.int32)
    for ch in range(BPW // CH):

        def body(t, carry):
            tid = plsc.load_gather(ids_v, [zero16 + (ch * CH + t)])
            off = tid * TW + lanes
            for c in range(D // L):
                zqb[t, pl.ds(c * L, L)] = plsc.load_gather(
                    tab_v, [off + (c * L)])
            xrb[t, pl.ds(0, L)] = plsc.load_gather(tab_v, [off + D])
            return carry

        lax.fori_loop(0, CH, body, 0)
        pltpu.sync_copy(zqb, zq_hbm.at[pl.ds(base + ch * CH, CH)])
        pltpu.sync_copy(xrb, xr_hbm.at[pl.ds(base + ch * CH, CH)])


def kernel(x, W_enc, b_enc, codebook, W_dec, b_dec):
    wd_pad = jnp.zeros((D, DP), jnp.float32).at[:, :D_IN].set(W_dec)
    bd_pad = jnp.zeros((1, DP), jnp.float32).at[0, :D_IN].set(b_dec)
    z_e, ids2d, tab = _tc_call(
        x, W_enc, b_enc.reshape(1, D), codebook.T, codebook, wd_pad, bd_pad)
    z_q, xr = _sc_gather(tab.reshape(K * TW), ids2d.reshape(T))
    return (xr[:, :D_IN], z_e, z_q)


# speedup vs baseline: 2.4374x; 1.0065x over previous
import functools

import jax
import jax.numpy as jnp
from jax import lax
from jax.experimental import pallas as pl
from jax.experimental.pallas import tpu as pltpu
from jax.experimental.pallas import tpu_sc as plsc

T = 32768
D_IN = 12
K = 1024
D = 64
DP = 16
TW = D + DP
TB = 1024
NC = 2
NS = 16
NW = NC * NS
BPW = T // NW
CH = 128
L = 16


def _prep_body(cbt_ref, cb_ref, wd_ref, bd_ref, c2_ref, tab_ref):
    cbt = cbt_ref[...]
    c2_ref[...] = jnp.sum(cbt * cbt, axis=0, keepdims=True)
    cb = cb_ref[...]
    tab_ref[:, 0:D] = cb
    tab_ref[:, D:TW] = jnp.dot(cb, wd_ref[...],
                               preferred_element_type=jnp.float32) + bd_ref[...]


_prep_call = pl.pallas_call(
    _prep_body,
    out_shape=[
        jax.ShapeDtypeStruct((1, K), jnp.float32),
        jax.ShapeDtypeStruct((K, TW), jnp.float32),
    ],
)


def _tc_body(x_ref, wenc_ref, benc_ref, cbt_ref, c2_ref, ze_ref, ids_ref):
    x = x_ref[...]
    z_e = jnp.dot(x, wenc_ref[...],
                  preferred_element_type=jnp.float32) + benc_ref[...]
    ze_ref[...] = z_e

    z2 = jnp.sum(z_e * z_e, axis=1, keepdims=True)
    zc = jnp.dot(z_e, cbt_ref[...],
                 preferred_element_type=jnp.float32)
    d2 = (z2 - 2.0 * zc) + c2_ref[...]


    ids_ref[...] = jnp.argmin(d2, axis=1).astype(jnp.int32)[:, None]


_tc_call = pl.pallas_call(
    _tc_body,
    grid=(T // TB,),
    in_specs=[
        pl.BlockSpec((TB, D_IN), lambda i: (i, 0)),
        pl.BlockSpec((D_IN, D), lambda i: (0, 0)),
        pl.BlockSpec((1, D), lambda i: (0, 0)),
        pl.BlockSpec((D, K), lambda i: (0, 0)),
        pl.BlockSpec((1, K), lambda i: (0, 0)),
    ],
    out_specs=[
        pl.BlockSpec((TB, D), lambda i: (i, 0)),
        pl.BlockSpec((TB, 1), lambda i: (i, 0)),
    ],
    out_shape=[
        jax.ShapeDtypeStruct((T, D), jnp.float32),
        jax.ShapeDtypeStruct((T, 1), jnp.int32),
    ],
)


@functools.partial(
    pl.kernel,
    out_type=(jax.ShapeDtypeStruct((T, D), jnp.float32),
              jax.ShapeDtypeStruct((T, DP), jnp.float32)),
    mesh=plsc.VectorSubcoreMesh(core_axis_name="c", subcore_axis_name="s"),
    scratch_types=[
        pltpu.VMEM((K * TW,), jnp.float32),
        pltpu.VMEM((BPW,), jnp.int32),
        pltpu.VMEM((CH, D), jnp.float32),
        pltpu.VMEM((CH, DP), jnp.float32),
    ],
    compiler_params=pltpu.CompilerParams(needs_layout_passes=False),
)
def _sc_gather(tab_hbm, ids_hbm, zq_hbm, xr_hbm, tab_v, ids_v, zqb, xrb):
    wid = lax.axis_index("s") * NC + lax.axis_index("c")
    base = wid * BPW
    pltpu.sync_copy(tab_hbm, tab_v)
    pltpu.sync_copy(ids_hbm.at[pl.ds(base, BPW)], ids_v)
    lanes = lax.iota(jnp.int32, L)
    zero16 = jnp.zeros((L,), jnp.int32)
    for ch in range(BPW // CH):

        def body(t, carry):
            tid = plsc.load_gather(ids_v, [zero16 + (ch * CH + t)])
            off = tid * TW + lanes
            for c in range(D // L):
                zqb[t, pl.ds(c * L, L)] = plsc.load_gather(
                    tab_v, [off + (c * L)])
            xrb[t, pl.ds(0, L)] = plsc.load_gather(tab_v, [off + D])
            return carry

        lax.fori_loop(0, CH, body, 0)
        pltpu.sync_copy(zqb, zq_hbm.at[pl.ds(base + ch * CH, CH)])
        pltpu.sync_copy(xrb, xr_hbm.at[pl.ds(base + ch * CH, CH)])


def kernel(x, W_enc, b_enc, codebook, W_dec, b_dec):
    wd_pad = jnp.zeros((D, DP), jnp.float32).at[:, :D_IN].set(W_dec)
    bd_pad = jnp.zeros((1, DP), jnp.float32).at[0, :D_IN].set(b_dec)
    cbt = codebook.T
    c2, tab = _prep_call(cbt, codebook, wd_pad, bd_pad)
    z_e, ids2d = _tc_call(x, W_enc, b_enc.reshape(1, D), cbt, c2)
    z_q, xr = _sc_gather(tab.reshape(K * TW), ids2d.reshape(T))
    return (xr[:, :D_IN], z_e, z_q)


# speedup vs baseline: 2.7510x; 1.1287x over previous
import functools

import jax
import jax.numpy as jnp
from jax import lax
from jax.experimental import pallas as pl
from jax.experimental.pallas import tpu as pltpu
from jax.experimental.pallas import tpu_sc as plsc

T = 32768
D_IN = 12
K = 1024
D = 64
DP = 16
TW = D + DP
TB = 1024
NC = 2
NS = 16
NW = NC * NS
BPW = T // NW
CH = 64
L = 16


def _prep_body(cbt_ref, cb_ref, wd_ref, bd_ref, c2_ref, tab_ref):
    cbt = cbt_ref[...]
    c2_ref[...] = jnp.sum(cbt * cbt, axis=0, keepdims=True)
    cb = cb_ref[...]
    tab_ref[:, 0:D] = cb
    tab_ref[:, D:TW] = jnp.dot(cb, wd_ref[...],
                               preferred_element_type=jnp.float32) + bd_ref[...]


_prep_call = pl.pallas_call(
    _prep_body,
    out_shape=[
        jax.ShapeDtypeStruct((1, K), jnp.float32),
        jax.ShapeDtypeStruct((K, TW), jnp.float32),
    ],
)


def _tc_body(x_ref, wenc_ref, benc_ref, cbt_ref, c2_ref, ze_ref, ids_ref):
    x = x_ref[...]
    z_e = jnp.dot(x, wenc_ref[...],
                  preferred_element_type=jnp.float32) + benc_ref[...]
    ze_ref[...] = z_e

    z2 = jnp.sum(z_e * z_e, axis=1, keepdims=True)
    zc = jnp.dot(z_e, cbt_ref[...],
                 preferred_element_type=jnp.float32)
    d2 = (z2 - 2.0 * zc) + c2_ref[...]


    ids_ref[...] = jnp.argmin(d2, axis=1).astype(jnp.int32)


_tc_call = pl.pallas_call(
    _tc_body,
    grid=(T // TB,),
    in_specs=[
        pl.BlockSpec((TB, D_IN), lambda i: (i, 0)),
        pl.BlockSpec((D_IN, D), lambda i: (0, 0)),
        pl.BlockSpec((1, D), lambda i: (0, 0)),
        pl.BlockSpec((D, K), lambda i: (0, 0)),
        pl.BlockSpec((1, K), lambda i: (0, 0)),
    ],
    out_specs=[
        pl.BlockSpec((TB, D), lambda i: (i, 0)),
        pl.BlockSpec((TB,), lambda i: (i,)),
    ],
    out_shape=[
        jax.ShapeDtypeStruct((T, D), jnp.float32),
        jax.ShapeDtypeStruct((T,), jnp.int32),
    ],
)


@functools.partial(
    pl.kernel,
    out_type=(jax.ShapeDtypeStruct((T, D), jnp.float32),
              jax.ShapeDtypeStruct((T, DP), jnp.float32)),
    mesh=plsc.VectorSubcoreMesh(core_axis_name="c", subcore_axis_name="s"),
    scratch_types=[
        pltpu.VMEM((K * TW,), jnp.float32),
        pltpu.VMEM((BPW,), jnp.int32),
        pltpu.VMEM((2, CH, D), jnp.float32),
        pltpu.VMEM((2, CH, DP), jnp.float32),
        pltpu.SemaphoreType.DMA,
    ],
    compiler_params=pltpu.CompilerParams(needs_layout_passes=False),
)
def _sc_gather(tab_hbm, ids_hbm, zq_hbm, xr_hbm, tab_v, ids_v, zqb, xrb, semw):
    wid = lax.axis_index("s") * NC + lax.axis_index("c")
    base = wid * BPW
    pltpu.sync_copy(tab_hbm, tab_v)
    pltpu.sync_copy(ids_hbm.at[pl.ds(base, BPW)], ids_v)
    lanes = lax.iota(jnp.int32, L)
    zero16 = jnp.zeros((L,), jnp.int32)
    nch = BPW // CH

    def fill(ch, b):
        @plsc.parallel_loop(0, CH, unroll=4)
        def _(t):
            tid = plsc.load_gather(ids_v, [zero16 + (ch * CH + t)])
            off = tid * TW + lanes
            for c in range(D // L):
                zqb[b, t, pl.ds(c * L, L)] = plsc.load_gather(
                    tab_v, [off + (c * L)])
            xrb[b, t, pl.ds(0, L)] = plsc.load_gather(tab_v, [off + D])

    def write_start(ch, b):
        pltpu.async_copy(zqb.at[b], zq_hbm.at[pl.ds(base + ch * CH, CH)], semw)
        pltpu.async_copy(xrb.at[b], xr_hbm.at[pl.ds(base + ch * CH, CH)], semw)

    def write_wait(ch, b):
        pltpu.make_async_copy(
            zqb.at[b], zq_hbm.at[pl.ds(base + ch * CH, CH)], semw).wait()
        pltpu.make_async_copy(
            xrb.at[b], xr_hbm.at[pl.ds(base + ch * CH, CH)], semw).wait()

    for ch in range(nch):
        b = ch % 2
        if ch >= 2:
            write_wait(ch - 2, b)
        fill(ch, b)
        write_start(ch, b)
    write_wait(nch - 2, nch % 2)
    write_wait(nch - 1, (nch - 1) % 2)


def kernel(x, W_enc, b_enc, codebook, W_dec, b_dec):
    wd_pad = jnp.zeros((D, DP), jnp.float32).at[:, :D_IN].set(W_dec)
    bd_pad = jnp.zeros((1, DP), jnp.float32).at[0, :D_IN].set(b_dec)
    cbt = codebook.T
    c2, tab = _prep_call(cbt, codebook, wd_pad, bd_pad)
    z_e, ids = _tc_call(x, W_enc, b_enc.reshape(1, D), cbt, c2)
    z_q, xr = _sc_gather(tab.reshape(K * TW), ids)
    return (xr[:, :D_IN], z_e, z_q)
